# baseline (speedup 1.0000x reference)
import functools

import jax
import jax.numpy as jnp
from jax.experimental import pallas as pl

F32 = jnp.float32
HI = jax.lax.Precision.HIGHEST


def _dot(a, b):
    return jnp.dot(a, b, preferred_element_type=F32)


def _xw3(main_ref, halo_ref, wo, relu):
    xp = jnp.concatenate([main_ref[0], halo_ref[0]], axis=0)
    if relu:
        xp = jnp.maximum(xp, 0.0)
    xw = jnp.concatenate([xp[:, u:u + wo, :] for u in range(3)], axis=-1)
    return xp, xw


def _in_specs_halo(th, wp, c):
    hb = th // 2
    return [
        pl.BlockSpec((1, th, wp, c), lambda n, i: (n, i, 0, 0)),
        pl.BlockSpec((1, 2, wp, c),
                     lambda n, i: (n, (i + 1) * hb, 0, 0)),
    ]


def _conv3_body(xm_ref, xh_ref, w_ref, b_ref, o_ref, *, th, Wo, Co,
                pre_relu, post_relu):
    _, xw = _xw3(xm_ref, xh_ref, Wo, pre_relu)
    acc = None
    for t in range(3):
        xs = xw[t:t + th].reshape(th * Wo, xw.shape[-1])
        part = _dot(xs, w_ref[t])
        acc = part if acc is None else acc + part
    acc = acc + b_ref[...]
    if post_relu:
        acc = jnp.maximum(acc, 0.0)
    o_ref[0] = acc.reshape(th, Wo, Co)


def _conv3(x, wt, b, th, *, pre_relu=False, post_relu=False):
    n, hp, wp, ci = x.shape
    h, wo = hp - 2, wp - 2
    co = wt.shape[-1]
    body = functools.partial(_conv3_body, th=th, Wo=wo, Co=co,
                             pre_relu=pre_relu, post_relu=post_relu)
    return pl.pallas_call(
        body,
        grid=(n, h // th),
        in_specs=_in_specs_halo(th, wp, ci) + [
            pl.BlockSpec((3, 3 * ci, co), lambda n, i: (0, 0, 0)),
            pl.BlockSpec((1, co), lambda n, i: (0, 0)),
        ],
        out_specs=pl.BlockSpec((1, th, wo, co), lambda n, i: (n, i, 0, 0)),
        out_shape=jax.ShapeDtypeStruct((n, h, wo, co), F32),
    )(x, x, wt, b.reshape(1, co))


def _resblock_body(xm_ref, xh_ref, w1_ref, b1_ref, w2_ref, b2_ref, o_ref,
                   *, th, W, C):
    xp, xw = _xw3(xm_ref, xh_ref, W, True)
    acc = None
    for t in range(3):
        xs = xw[t:t + th].reshape(th * W, 3 * C)
        part = _dot(xs, w1_ref[t])
        acc = part if acc is None else acc + part
    h = jnp.maximum(acc + b1_ref[...], 0.0)
    h2 = _dot(h, w2_ref[...]) + b2_ref[...]
    xc = xm_ref[0][1:, 1:1 + W, :]
    xc = jnp.concatenate([xc, xh_ref[0, 0:1, 1:1 + W, :]], axis=0)
    o_ref[0] = (xc.reshape(th * W, C) + h2).reshape(th, W, C)


def _resblock(x, w1, b1, w2, b2, th):
    n, hp, wp, c = x.shape
    h, wo = hp - 2, wp - 2
    r = w1.shape[-1]
    body = functools.partial(_resblock_body, th=th, W=wo, C=c)
    return pl.pallas_call(
        body,
        grid=(n, h // th),
        in_specs=_in_specs_halo(th, wp, c) + [
            pl.BlockSpec((3, 3 * c, r), lambda n, i: (0, 0, 0)),
            pl.BlockSpec((1, r), lambda n, i: (0, 0)),
            pl.BlockSpec((r, c), lambda n, i: (0, 0)),
            pl.BlockSpec((1, c), lambda n, i: (0, 0)),
        ],
        out_specs=pl.BlockSpec((1, th, wo, c), lambda n, i: (n, i, 0, 0)),
        out_shape=jax.ShapeDtypeStruct((n, h, wo, c), F32),
    )(x, x, w1, b1.reshape(1, r), w2, b2.reshape(1, c))


def _vq_body(q_ref, sc_ref, cols_ref, embed_t_ref,
             quant_ref, diff_ref, *, R, K, inv_n):
    i = pl.program_id(0)
    q = q_ref[...]
    rows = jnp.sum(q * q, axis=1, keepdims=True)
    dist = rows - 2.0 * sc_ref[...] + cols_ref[...]
    ind = jnp.argmin(dist, axis=1).reshape(R, 1)
    onehot = (ind == jax.lax.broadcasted_iota(jnp.int32, (R, K), 1)
              ).astype(F32)
    quant = jnp.dot(onehot, embed_t_ref[...],
                    preferred_element_type=F32, precision=HI)
    quant_ref[...] = quant
    part = (jnp.sum((quant - q) ** 2) * inv_n).reshape(1, 1)

    @pl.when(i == 0)
    def _():
        diff_ref[...] = part

    @pl.when(i != 0)
    def _():
        diff_ref[...] += part


def _vq(q_flat, sc, cols, embed_t, n_blocks):
    rows, e = q_flat.shape
    k = sc.shape[1]
    r = rows // n_blocks
    body = functools.partial(_vq_body, R=r, K=k, inv_n=1.0 / (rows * e))
    return pl.pallas_call(
        body,
        grid=(n_blocks,),
        in_specs=[
            pl.BlockSpec((r, e), lambda i: (i, 0)),
            pl.BlockSpec((r, k), lambda i: (i, 0)),
            pl.BlockSpec((1, k), lambda i: (0, 0)),
            pl.BlockSpec((k, e), lambda i: (0, 0)),
        ],
        out_specs=[
            pl.BlockSpec((r, e), lambda i: (i, 0)),
            pl.BlockSpec((1, 1), lambda i: (0, 0)),
        ],
        out_shape=[
            jax.ShapeDtypeStruct((rows, e), F32),
            jax.ShapeDtypeStruct((1, 1), F32),
        ],
    )(q_flat, sc, cols, embed_t)


def _dtrans_body(xm_ref, xh_ref, w_ref, b_ref, o_ref, *, th, W, Co,
                 pre_relu, post_relu):
    _, xw = _xw3(xm_ref, xh_ref, W, pre_relu)
    xs = [xw[oy:oy + th].reshape(th * W, xw.shape[-1]) for oy in range(3)]
    outs = []
    for qy in range(2):
        for qx in range(2):
            acc = None
            for ty in range(2):
                part = _dot(xs[qy + ty], w_ref[qy, qx, ty])
                acc = part if acc is None else acc + part
            acc = acc + b_ref[...]
            if post_relu:
                acc = jnp.maximum(acc, 0.0)
            outs.append(acc)
    o_ref[0] = jnp.concatenate(outs, axis=1).reshape(th, W, 4 * Co)


def _dtrans(x, w, b, th, *, pre_relu, post_relu):
    n, hp, wp, ci = x.shape
    h, wo = hp - 2, wp - 2
    co = w.shape[-1]
    wq = jnp.zeros((2, 2, 2, 3 * ci, co), F32)
    for qy in range(2):
        for qx in range(2):
            for ty in range(2):
                for tx in range(2):
                    ky, kx = 3 - qy - 2 * ty, 3 - qx - 2 * tx
                    ox = qx + tx
                    wq = wq.at[qy, qx, ty,
                               ox * ci:(ox + 1) * ci].set(w[ky, kx])
    body = functools.partial(_dtrans_body, th=th, W=wo, Co=co,
                             pre_relu=pre_relu, post_relu=post_relu)
    return pl.pallas_call(
        body,
        grid=(n, h // th),
        in_specs=_in_specs_halo(th, wp, ci) + [
            pl.BlockSpec((2, 2, 2, 3 * ci, co),
                         lambda n, i: (0, 0, 0, 0, 0)),
            pl.BlockSpec((1, co), lambda n, i: (0, 0)),
        ],
        out_specs=pl.BlockSpec((1, th, wo, 4 * co),
                               lambda n, i: (n, i, 0, 0)),
        out_shape=jax.ShapeDtypeStruct((n, h, wo, 4 * co), F32),
    )(x, x, wq, b.reshape(1, co))


def _pad1(x):
    return jnp.pad(x, ((0, 0), (1, 1), (1, 1), (0, 0)))


def _d2s(x):
    n, h, w, c4 = x.shape
    c = c4 // 4
    return (x.reshape(n, h, w, 2, 2, c).transpose(0, 1, 3, 2, 4, 5)
            .reshape(n, 2 * h, 2 * w, c))


def _w_conv3(w):
    k = jnp.transpose(w, (2, 3, 1, 0))
    return k.reshape(3, 3 * k.shape[2], k.shape[3])


def _w_dtrans(w):
    return jnp.transpose(w, (2, 3, 0, 1))


def _conv_nchw(x, w, b, stride, pad):
    out = jax.lax.conv_general_dilated(
        x, w, (stride, stride), [(pad, pad), (pad, pad)],
        dimension_numbers=('NCHW', 'OIHW', 'NCHW'))
    return out + b[None, :, None, None]


def _res_block_nchw(x, w1, b1, w2, b2):
    out = jax.nn.relu(x)
    out = _conv_nchw(out, w1, b1, 1, 1)
    out = jax.nn.relu(out)
    out = _conv_nchw(out, w2, b2, 1, 0)
    return x + out


def kernel(input, e1w, e1b, e2w, e2b, e3w, e3b,
           er1w1, er1b1, er1w2, er1b2, er2w1, er2b1, er2w2, er2b2,
           qw, qb, embed, d1w, d1b,
           dr1w1, dr1b1, dr1w2, dr1b2, dr2w1, dr2b1, dr2w2, dr2b2,
           dt1w, dt1b, dt2w, dt2b):
    n = input.shape[0]

    h = jax.nn.relu(_conv_nchw(input, e1w, e1b, 2, 1))
    h = jax.nn.relu(_conv_nchw(h, e2w, e2b, 2, 1))
    h = _conv_nchw(h, e3w, e3b, 1, 1)
    h = _res_block_nchw(h, er1w1, er1b1, er1w2, er1b2)
    h = _res_block_nchw(h, er2w1, er2b1, er2w2, er2b2)
    h = jax.nn.relu(h)
    q = _conv_nchw(h, qw, qb, 1, 0)
    q = jnp.transpose(q, (0, 2, 3, 1))

    e = embed.shape[0]
    q_flat = q.reshape(-1, e)
    cols = (embed ** 2).sum(0)[None]
    sc = q_flat @ embed
    quant_flat, diff = _vq(q_flat, sc, cols, embed.T, 16)
    quant = quant_flat.reshape(n, 96, 96, e)

    d = _conv3(_pad1(quant), _w_conv3(d1w), d1b, 24)
    d = _resblock(_pad1(d), _w_conv3(dr1w1), dr1b1,
                  dr1w2[:, :, 0, 0].T, dr1b2, 24)
    d = _resblock(_pad1(d), _w_conv3(dr2w1), dr2b1,
                  dr2w2[:, :, 0, 0].T, dr2b2, 24)
    d = _dtrans(_pad1(d), _w_dtrans(dt1w), dt1b, 24,
                pre_relu=True, post_relu=True)
    d = _d2s(d)
    d = _dtrans(_pad1(d), _w_dtrans(dt2w), dt2b, 24,
                pre_relu=False, post_relu=False)
    d = _d2s(d)
    d = jnp.transpose(d, (0, 3, 1, 2))
    return (d, diff.reshape(1))


# speedup vs baseline: 1.0073x; 1.0073x over previous
import functools

import jax
import jax.numpy as jnp
from jax.experimental import pallas as pl

F32 = jnp.float32
HI = jax.lax.Precision.HIGHEST


def _dot(a, b):
    return jnp.dot(a, b, preferred_element_type=F32)


def _xw3(main_ref, halo_ref, wo, relu):
    xp = jnp.concatenate([main_ref[0], halo_ref[0]], axis=0)
    if relu:
        xp = jnp.maximum(xp, 0.0)
    xw = jnp.concatenate([xp[:, u:u + wo, :] for u in range(3)], axis=-1)
    return xp, xw


def _in_specs_halo(th, wp, c):
    hb = th // 2
    return [
        pl.BlockSpec((1, th, wp, c), lambda n, i: (n, i, 0, 0)),
        pl.BlockSpec((1, 2, wp, c),
                     lambda n, i: (n, (i + 1) * hb, 0, 0)),
    ]


def _conv3_body(xm_ref, xh_ref, w_ref, b_ref, o_ref, *, th, Wo, Co,
                pre_relu, post_relu):
    _, xw = _xw3(xm_ref, xh_ref, Wo, pre_relu)
    acc = None
    for t in range(3):
        xs = xw[t:t + th].reshape(th * Wo, xw.shape[-1])
        part = _dot(xs, w_ref[t])
        acc = part if acc is None else acc + part
    acc = acc + b_ref[...]
    if post_relu:
        acc = jnp.maximum(acc, 0.0)
    o_ref[0] = acc.reshape(th, Wo, Co)


def _conv3(x, wt, b, th, *, pre_relu=False, post_relu=False):
    n, hp, wp, ci = x.shape
    h, wo = hp - 2, wp - 2
    co = wt.shape[-1]
    body = functools.partial(_conv3_body, th=th, Wo=wo, Co=co,
                             pre_relu=pre_relu, post_relu=post_relu)
    return pl.pallas_call(
        body,
        grid=(n, h // th),
        in_specs=_in_specs_halo(th, wp, ci) + [
            pl.BlockSpec((3, 3 * ci, co), lambda n, i: (0, 0, 0)),
            pl.BlockSpec((1, co), lambda n, i: (0, 0)),
        ],
        out_specs=pl.BlockSpec((1, th, wo, co), lambda n, i: (n, i, 0, 0)),
        out_shape=jax.ShapeDtypeStruct((n, h, wo, co), F32),
    )(x, x, wt, b.reshape(1, co))


def _resblock_body(xm_ref, xh_ref, w1_ref, b1_ref, w2_ref, b2_ref, o_ref,
                   *, th, W, C):
    xp, xw = _xw3(xm_ref, xh_ref, W, True)
    acc = None
    for t in range(3):
        xs = xw[t:t + th].reshape(th * W, 3 * C)
        part = _dot(xs, w1_ref[t])
        acc = part if acc is None else acc + part
    h = jnp.maximum(acc + b1_ref[...], 0.0)
    h2 = _dot(h, w2_ref[...]) + b2_ref[...]
    xc = xm_ref[0][1:, 1:1 + W, :]
    xc = jnp.concatenate([xc, xh_ref[0, 0:1, 1:1 + W, :]], axis=0)
    o_ref[0] = (xc.reshape(th * W, C) + h2).reshape(th, W, C)


def _resblock(x, w1, b1, w2, b2, th):
    n, hp, wp, c = x.shape
    h, wo = hp - 2, wp - 2
    r = w1.shape[-1]
    body = functools.partial(_resblock_body, th=th, W=wo, C=c)
    return pl.pallas_call(
        body,
        grid=(n, h // th),
        in_specs=_in_specs_halo(th, wp, c) + [
            pl.BlockSpec((3, 3 * c, r), lambda n, i: (0, 0, 0)),
            pl.BlockSpec((1, r), lambda n, i: (0, 0)),
            pl.BlockSpec((r, c), lambda n, i: (0, 0)),
            pl.BlockSpec((1, c), lambda n, i: (0, 0)),
        ],
        out_specs=pl.BlockSpec((1, th, wo, c), lambda n, i: (n, i, 0, 0)),
        out_shape=jax.ShapeDtypeStruct((n, h, wo, c), F32),
    )(x, x, w1, b1.reshape(1, r), w2, b2.reshape(1, c))


def _vq_body(q_ref, sc_ref, cols_ref, embed_t_ref,
             quant_ref, diff_ref, *, R, K, inv_n):
    i = pl.program_id(0)
    q = q_ref[...]
    rows = jnp.sum(q * q, axis=1, keepdims=True)
    dist = rows - 2.0 * sc_ref[...] + cols_ref[...]
    ind = jnp.argmin(dist, axis=1).reshape(R, 1)
    onehot = (ind == jax.lax.broadcasted_iota(jnp.int32, (R, K), 1)
              ).astype(F32)
    quant = jnp.dot(onehot, embed_t_ref[...],
                    preferred_element_type=F32, precision=HI)
    quant_ref[...] = quant
    part = (jnp.sum((quant - q) ** 2) * inv_n).reshape(1, 1)

    @pl.when(i == 0)
    def _():
        diff_ref[...] = part

    @pl.when(i != 0)
    def _():
        diff_ref[...] += part


def _vq(q_flat, sc, cols, embed_t, n_blocks):
    rows, e = q_flat.shape
    k = sc.shape[1]
    r = rows // n_blocks
    body = functools.partial(_vq_body, R=r, K=k, inv_n=1.0 / (rows * e))
    return pl.pallas_call(
        body,
        grid=(n_blocks,),
        in_specs=[
            pl.BlockSpec((r, e), lambda i: (i, 0)),
            pl.BlockSpec((r, k), lambda i: (i, 0)),
            pl.BlockSpec((1, k), lambda i: (0, 0)),
            pl.BlockSpec((k, e), lambda i: (0, 0)),
        ],
        out_specs=[
            pl.BlockSpec((r, e), lambda i: (i, 0)),
            pl.BlockSpec((1, 1), lambda i: (0, 0)),
        ],
        out_shape=[
            jax.ShapeDtypeStruct((rows, e), F32),
            jax.ShapeDtypeStruct((1, 1), F32),
        ],
    )(q_flat, sc, cols, embed_t)


def _dtrans_body(xm_ref, xh_ref, w_ref, b_ref, o_ref, *, th, W, Co,
                 pre_relu, post_relu):
    _, xw = _xw3(xm_ref, xh_ref, W, pre_relu)
    xs = [xw[oy:oy + th].reshape(th * W, xw.shape[-1]) for oy in range(3)]
    outs = []
    for qy in range(2):
        for qx in range(2):
            acc = None
            for ty in range(2):
                part = _dot(xs[qy + ty], w_ref[qy, qx, ty])
                acc = part if acc is None else acc + part
            acc = acc + b_ref[...]
            if post_relu:
                acc = jnp.maximum(acc, 0.0)
            outs.append(acc)
    o_ref[0] = jnp.concatenate(outs, axis=1).reshape(th, W, 4 * Co)


def _dtrans(x, w, b, th, *, pre_relu, post_relu):
    n, hp, wp, ci = x.shape
    h, wo = hp - 2, wp - 2
    co = w.shape[-1]
    wq = jnp.zeros((2, 2, 2, 3 * ci, co), F32)
    for qy in range(2):
        for qx in range(2):
            for ty in range(2):
                for tx in range(2):
                    ky, kx = 3 - qy - 2 * ty, 3 - qx - 2 * tx
                    ox = qx + tx
                    wq = wq.at[qy, qx, ty,
                               ox * ci:(ox + 1) * ci].set(w[ky, kx])
    body = functools.partial(_dtrans_body, th=th, W=wo, Co=co,
                             pre_relu=pre_relu, post_relu=post_relu)
    return pl.pallas_call(
        body,
        grid=(n, h // th),
        in_specs=_in_specs_halo(th, wp, ci) + [
            pl.BlockSpec((2, 2, 2, 3 * ci, co),
                         lambda n, i: (0, 0, 0, 0, 0)),
            pl.BlockSpec((1, co), lambda n, i: (0, 0)),
        ],
        out_specs=pl.BlockSpec((1, th, wo, 4 * co),
                               lambda n, i: (n, i, 0, 0)),
        out_shape=jax.ShapeDtypeStruct((n, h, wo, 4 * co), F32),
    )(x, x, wq, b.reshape(1, co))


def _pad1(x):
    return jnp.pad(x, ((0, 0), (1, 1), (1, 1), (0, 0)))


def _d2s(x):
    n, h, w, c4 = x.shape
    c = c4 // 4
    return (x.reshape(n, h, w, 2, 2, c).transpose(0, 1, 3, 2, 4, 5)
            .reshape(n, 2 * h, 2 * w, c))


def _w_conv3(w):
    k = jnp.transpose(w, (2, 3, 1, 0))
    return k.reshape(3, 3 * k.shape[2], k.shape[3])


def _w_dtrans(w):
    return jnp.transpose(w, (2, 3, 0, 1))


def _conv_nchw(x, w, b, stride, pad):
    out = jax.lax.conv_general_dilated(
        x, w, (stride, stride), [(pad, pad), (pad, pad)],
        dimension_numbers=('NCHW', 'OIHW', 'NCHW'))
    return out + b[None, :, None, None]


def _res_block_nchw(x, w1, b1, w2, b2):
    out = jax.nn.relu(x)
    out = _conv_nchw(out, w1, b1, 1, 1)
    out = jax.nn.relu(out)
    out = _conv_nchw(out, w2, b2, 1, 0)
    return x + out


def kernel(input, e1w, e1b, e2w, e2b, e3w, e3b,
           er1w1, er1b1, er1w2, er1b2, er2w1, er2b1, er2w2, er2b2,
           qw, qb, embed, d1w, d1b,
           dr1w1, dr1b1, dr1w2, dr1b2, dr2w1, dr2b1, dr2w2, dr2b2,
           dt1w, dt1b, dt2w, dt2b):
    n = input.shape[0]

    h = jax.nn.relu(_conv_nchw(input, e1w, e1b, 2, 1))
    h = jax.nn.relu(_conv_nchw(h, e2w, e2b, 2, 1))
    h = _conv_nchw(h, e3w, e3b, 1, 1)
    h = _res_block_nchw(h, er1w1, er1b1, er1w2, er1b2)
    h = _res_block_nchw(h, er2w1, er2b1, er2w2, er2b2)
    h = jax.nn.relu(h)
    q = _conv_nchw(h, qw, qb, 1, 0)
    q = jnp.transpose(q, (0, 2, 3, 1))

    e = embed.shape[0]
    q_flat = q.reshape(-1, e)
    cols = (embed ** 2).sum(0)[None]
    sc = q_flat @ embed
    quant_flat, diff = _vq(q_flat, sc, cols, embed.T, 16)
    quant = quant_flat.reshape(n, 96, 96, e)

    d = _conv3(_pad1(quant), _w_conv3(d1w), d1b, 48)
    d = _resblock(_pad1(d), _w_conv3(dr1w1), dr1b1,
                  dr1w2[:, :, 0, 0].T, dr1b2, 48)
    d = _resblock(_pad1(d), _w_conv3(dr2w1), dr2b1,
                  dr2w2[:, :, 0, 0].T, dr2b2, 48)
    d = _dtrans(_pad1(d), _w_dtrans(dt1w), dt1b, 32,
                pre_relu=True, post_relu=True)
    d = _d2s(d)
    d = _dtrans(_pad1(d), _w_dtrans(dt2w), dt2b, 32,
                pre_relu=False, post_relu=False)
    d = _d2s(d)
    d = jnp.transpose(d, (0, 3, 1, 2))
    return (d, diff.reshape(1))


# speedup vs baseline: 1.0345x; 1.0270x over previous
import functools

import jax
import jax.numpy as jnp
from jax.experimental import pallas as pl

F32 = jnp.float32
HI = jax.lax.Precision.HIGHEST


def _dot(a, b):
    return jnp.dot(a, b, preferred_element_type=F32)


def _xw3(main_ref, halo_ref, wo, relu):
    xp = jnp.concatenate([main_ref[0], halo_ref[0]], axis=0)
    if relu:
        xp = jnp.maximum(xp, 0.0)
    xw = jnp.concatenate([xp[:, u:u + wo, :] for u in range(3)], axis=-1)
    return xp, xw


def _in_specs_halo(th, wp, c):
    hb = th // 2
    return [
        pl.BlockSpec((1, th, wp, c), lambda n, i: (n, i, 0, 0)),
        pl.BlockSpec((1, 2, wp, c),
                     lambda n, i: (n, (i + 1) * hb, 0, 0)),
    ]


def _conv3_body(xm_ref, xh_ref, w_ref, b_ref, o_ref, *, th, Wo, Co,
                pre_relu, post_relu):
    _, xw = _xw3(xm_ref, xh_ref, Wo, pre_relu)
    acc = None
    for t in range(3):
        xs = xw[t:t + th].reshape(th * Wo, xw.shape[-1])
        part = _dot(xs, w_ref[t])
        acc = part if acc is None else acc + part
    acc = acc + b_ref[...]
    if post_relu:
        acc = jnp.maximum(acc, 0.0)
    o_ref[0] = acc.reshape(th, Wo, Co)


def _conv3(x, wt, b, th, *, pre_relu=False, post_relu=False):
    n, hp, wp, ci = x.shape
    h, wo = hp - 2, wp - 2
    co = wt.shape[-1]
    body = functools.partial(_conv3_body, th=th, Wo=wo, Co=co,
                             pre_relu=pre_relu, post_relu=post_relu)
    return pl.pallas_call(
        body,
        grid=(n, h // th),
        in_specs=_in_specs_halo(th, wp, ci) + [
            pl.BlockSpec((3, 3 * ci, co), lambda n, i: (0, 0, 0)),
            pl.BlockSpec((1, co), lambda n, i: (0, 0)),
        ],
        out_specs=pl.BlockSpec((1, th, wo, co), lambda n, i: (n, i, 0, 0)),
        out_shape=jax.ShapeDtypeStruct((n, h, wo, co), F32),
    )(x, x, wt, b.reshape(1, co))


def _resblock_body(xm_ref, xh_ref, w1_ref, b1_ref, w2_ref, b2_ref, o_ref,
                   *, th, W, C):
    xp, xw = _xw3(xm_ref, xh_ref, W, True)
    acc = None
    for t in range(3):
        xs = xw[t:t + th].reshape(th * W, 3 * C)
        part = _dot(xs, w1_ref[t])
        acc = part if acc is None else acc + part
    h = jnp.maximum(acc + b1_ref[...], 0.0)
    h2 = _dot(h, w2_ref[...]) + b2_ref[...]
    xc = xm_ref[0][1:, 1:1 + W, :]
    xc = jnp.concatenate([xc, xh_ref[0, 0:1, 1:1 + W, :]], axis=0)
    o_ref[0] = (xc.reshape(th * W, C) + h2).reshape(th, W, C)


def _resblock(x, w1, b1, w2, b2, th):
    n, hp, wp, c = x.shape
    h, wo = hp - 2, wp - 2
    r = w1.shape[-1]
    body = functools.partial(_resblock_body, th=th, W=wo, C=c)
    return pl.pallas_call(
        body,
        grid=(n, h // th),
        in_specs=_in_specs_halo(th, wp, c) + [
            pl.BlockSpec((3, 3 * c, r), lambda n, i: (0, 0, 0)),
            pl.BlockSpec((1, r), lambda n, i: (0, 0)),
            pl.BlockSpec((r, c), lambda n, i: (0, 0)),
            pl.BlockSpec((1, c), lambda n, i: (0, 0)),
        ],
        out_specs=pl.BlockSpec((1, th, wo, c), lambda n, i: (n, i, 0, 0)),
        out_shape=jax.ShapeDtypeStruct((n, h, wo, c), F32),
    )(x, x, w1, b1.reshape(1, r), w2, b2.reshape(1, c))


def _vq_body(q_ref, sc_ref, cols_ref, et_hi_ref, et_lo_ref,
             quant_ref, diff_ref, *, R, K, inv_n):
    i = pl.program_id(0)
    q = q_ref[...]
    rows = jnp.sum(q * q, axis=1, keepdims=True)
    dist = rows - 2.0 * sc_ref[...] + cols_ref[...]
    ind = jnp.argmin(dist, axis=1).reshape(R, 1)
    onehot = (ind == jax.lax.broadcasted_iota(jnp.int32, (R, K), 1)
              ).astype(F32)
    quant = _dot(onehot, et_hi_ref[...]) + _dot(onehot, et_lo_ref[...])
    quant_ref[...] = quant
    part = (jnp.sum((quant - q) ** 2) * inv_n).reshape(1, 1)

    @pl.when(i == 0)
    def _():
        diff_ref[...] = part

    @pl.when(i != 0)
    def _():
        diff_ref[...] += part


def _vq(q_flat, sc, cols, embed_t, n_blocks):
    rows, e = q_flat.shape
    k = sc.shape[1]
    r = rows // n_blocks
    et_hi = jax.lax.bitcast_convert_type(
        jax.lax.bitcast_convert_type(embed_t, jnp.int32)
        & jnp.int32(-65536), F32)
    et_lo = embed_t - et_hi
    body = functools.partial(_vq_body, R=r, K=k, inv_n=1.0 / (rows * e))
    return pl.pallas_call(
        body,
        grid=(n_blocks,),
        in_specs=[
            pl.BlockSpec((r, e), lambda i: (i, 0)),
            pl.BlockSpec((r, k), lambda i: (i, 0)),
            pl.BlockSpec((1, k), lambda i: (0, 0)),
            pl.BlockSpec((k, e), lambda i: (0, 0)),
            pl.BlockSpec((k, e), lambda i: (0, 0)),
        ],
        out_specs=[
            pl.BlockSpec((r, e), lambda i: (i, 0)),
            pl.BlockSpec((1, 1), lambda i: (0, 0)),
        ],
        out_shape=[
            jax.ShapeDtypeStruct((rows, e), F32),
            jax.ShapeDtypeStruct((1, 1), F32),
        ],
    )(q_flat, sc, cols, et_hi, et_lo)


def _dtrans_body(xm_ref, xh_ref, w_ref, b_ref, o_ref, *, th, W, Co,
                 pre_relu, post_relu):
    _, xw = _xw3(xm_ref, xh_ref, W, pre_relu)
    xs = [xw[oy:oy + th].reshape(th * W, xw.shape[-1]) for oy in range(3)]
    outs = []
    for qy in range(2):
        for qx in range(2):
            acc = None
            for ty in range(2):
                part = _dot(xs[qy + ty], w_ref[qy, qx, ty])
                acc = part if acc is None else acc + part
            acc = acc + b_ref[...]
            if post_relu:
                acc = jnp.maximum(acc, 0.0)
            outs.append(acc)
    o_ref[0] = jnp.concatenate(outs, axis=1).reshape(th, W, 4 * Co)


def _dtrans(x, w, b, th, *, pre_relu, post_relu):
    n, hp, wp, ci = x.shape
    h, wo = hp - 2, wp - 2
    co = w.shape[-1]
    wq = jnp.zeros((2, 2, 2, 3 * ci, co), F32)
    for qy in range(2):
        for qx in range(2):
            for ty in range(2):
                for tx in range(2):
                    ky, kx = 3 - qy - 2 * ty, 3 - qx - 2 * tx
                    ox = qx + tx
                    wq = wq.at[qy, qx, ty,
                               ox * ci:(ox + 1) * ci].set(w[ky, kx])
    body = functools.partial(_dtrans_body, th=th, W=wo, Co=co,
                             pre_relu=pre_relu, post_relu=post_relu)
    return pl.pallas_call(
        body,
        grid=(n, h // th),
        in_specs=_in_specs_halo(th, wp, ci) + [
            pl.BlockSpec((2, 2, 2, 3 * ci, co),
                         lambda n, i: (0, 0, 0, 0, 0)),
            pl.BlockSpec((1, co), lambda n, i: (0, 0)),
        ],
        out_specs=pl.BlockSpec((1, th, wo, 4 * co),
                               lambda n, i: (n, i, 0, 0)),
        out_shape=jax.ShapeDtypeStruct((n, h, wo, 4 * co), F32),
    )(x, x, wq, b.reshape(1, co))


def _pad1(x):
    return jnp.pad(x, ((0, 0), (1, 1), (1, 1), (0, 0)))


def _d2s(x):
    n, h, w, c4 = x.shape
    c = c4 // 4
    return (x.reshape(n, h, w, 2, 2, c).transpose(0, 1, 3, 2, 4, 5)
            .reshape(n, 2 * h, 2 * w, c))


def _w_conv3(w):
    k = jnp.transpose(w, (2, 3, 1, 0))
    return k.reshape(3, 3 * k.shape[2], k.shape[3])


def _w_dtrans(w):
    return jnp.transpose(w, (2, 3, 0, 1))


def _conv_nchw(x, w, b, stride, pad):
    out = jax.lax.conv_general_dilated(
        x, w, (stride, stride), [(pad, pad), (pad, pad)],
        dimension_numbers=('NCHW', 'OIHW', 'NCHW'))
    return out + b[None, :, None, None]


def _res_block_nchw(x, w1, b1, w2, b2):
    out = jax.nn.relu(x)
    out = _conv_nchw(out, w1, b1, 1, 1)
    out = jax.nn.relu(out)
    out = _conv_nchw(out, w2, b2, 1, 0)
    return x + out


def kernel(input, e1w, e1b, e2w, e2b, e3w, e3b,
           er1w1, er1b1, er1w2, er1b2, er2w1, er2b1, er2w2, er2b2,
           qw, qb, embed, d1w, d1b,
           dr1w1, dr1b1, dr1w2, dr1b2, dr2w1, dr2b1, dr2w2, dr2b2,
           dt1w, dt1b, dt2w, dt2b):
    n = input.shape[0]

    h = jax.nn.relu(_conv_nchw(input, e1w, e1b, 2, 1))
    h = jax.nn.relu(_conv_nchw(h, e2w, e2b, 2, 1))
    h = _conv_nchw(h, e3w, e3b, 1, 1)
    h = _res_block_nchw(h, er1w1, er1b1, er1w2, er1b2)
    h = _res_block_nchw(h, er2w1, er2b1, er2w2, er2b2)
    h = jax.nn.relu(h)
    q = _conv_nchw(h, qw, qb, 1, 0)
    q = jnp.transpose(q, (0, 2, 3, 1))

    e = embed.shape[0]
    q_flat = q.reshape(-1, e)
    cols = (embed ** 2).sum(0)[None]
    sc = q_flat @ embed
    quant_flat, diff = _vq(q_flat, sc, cols, embed.T, 16)
    quant = quant_flat.reshape(n, 96, 96, e)

    d = _conv3(_pad1(quant), _w_conv3(d1w), d1b, 48)
    d = _resblock(_pad1(d), _w_conv3(dr1w1), dr1b1,
                  dr1w2[:, :, 0, 0].T, dr1b2, 48)
    d = _resblock(_pad1(d), _w_conv3(dr2w1), dr2b1,
                  dr2w2[:, :, 0, 0].T, dr2b2, 48)
    d = _dtrans(_pad1(d), _w_dtrans(dt1w), dt1b, 32,
                pre_relu=True, post_relu=True)
    d = _d2s(d)
    d = _dtrans(_pad1(d), _w_dtrans(dt2w), dt2b, 32,
                pre_relu=False, post_relu=False)
    d = _d2s(d)
    d = jnp.transpose(d, (0, 3, 1, 2))
    return (d, diff.reshape(1))
